# baseline (speedup 1.0000x reference)
import functools
import math

import jax
import jax.numpy as jnp
from jax import lax
from jax.experimental import pallas as pl
from jax.experimental.pallas import tpu as pltpu
from jax.experimental.pallas import tpu_sc as plsc

N = 10000
E = 100000
C = 32
H = 4
DH = C // H
NB = 16
HID = 64

HE = E // 2
NW = 32
CHUNK = 128
NCH = 13
EPW = NCH * CHUNK
EPH = NW * EPW
NP = 10240
RPS = NP // 16

EB = 1000
NBK = 1000

f32 = jnp.float32
bf16 = jnp.bfloat16
i32 = jnp.int32

GROUPS = ((0, 7), (7, 6))


def _sc_mesh():
    return plsc.VectorSubcoreMesh(core_axis_name="c", subcore_axis_name="s")


def _gather_body(nf_hbm, src3_hbm, dst3_hbm, xs_hbm, xd_hbm,
                 idx1, idx2, rows1, rows2, sem1, sem2):
    wid = lax.axis_index("s") * 2 + lax.axis_index("c")
    base = wid * EPW
    pltpu.sync_copy(src3_hbm.at[wid], idx1)
    pltpu.sync_copy(dst3_hbm.at[wid], idx2)
    for s0, gn in GROUPS:
        cps = []
        for j in range(gn):
            c = s0 + j
            cps.append(pltpu.async_copy(
                nf_hbm.at[idx1.at[c]],
                rows1.at[pl.ds(c * CHUNK, CHUNK)], sem1))
            cps.append(pltpu.async_copy(
                nf_hbm.at[idx2.at[c]],
                rows2.at[pl.ds(c * CHUNK, CHUNK)], sem2))
        for cp in cps:
            cp.wait()
    pltpu.sync_copy(rows1, xs_hbm.at[pl.ds(base, EPW)])
    pltpu.sync_copy(rows2, xd_hbm.at[pl.ds(base, EPW)])


def _scatter_body(pay_hbm, dst3_hbm, init_hbm, part_hbm, idxb, payb, shared, sem):
    cid = lax.axis_index("c")
    sid = lax.axis_index("s")
    wid = sid * 2 + cid
    r0 = sid * RPS
    pltpu.sync_copy(init_hbm.at[cid, pl.ds(r0, RPS)], shared.at[pl.ds(r0, RPS)])
    pltpu.sync_copy(dst3_hbm.at[wid], idxb)
    plsc.subcore_barrier()
    for s0, gn in GROUPS:
        pltpu.sync_copy(pay_hbm.at[pl.ds(wid * EPW + s0 * CHUNK, gn * CHUNK)],
                        payb.at[pl.ds(0, gn * CHUNK)])
        cps = []
        for j in range(gn):
            cps.append(pltpu.async_copy(
                payb.at[pl.ds(j * CHUNK, CHUNK)],
                shared.at[idxb.at[s0 + j]], sem, add=True))
        for cp in cps:
            cp.wait()
    plsc.subcore_barrier()
    pltpu.sync_copy(shared.at[pl.ds(r0, RPS)], part_hbm.at[cid, pl.ds(r0, RPS)])


def _bmm(a, b):
    return lax.dot_general(a.astype(bf16), b.astype(bf16),
                           (((1,), (0,)), ((), ())),
                           preferred_element_type=f32)


def _edge_body(emb_ref, sh_ref, xs_ref, xd_ref, wq_ref,
               wk1_ref, bk1_ref, wk2_ref, bk2_ref,
               wv1_ref, bv1_ref, wv2_ref, bv2_ref,
               tm_ref, sm_ref, bd_ref, s2_ref, e4_ref, out_ref):
    isc = 1.0 / math.sqrt(C)
    xs = xs_ref[...].astype(f32) * sh_ref[...]
    xd = xd_ref[...].astype(f32)
    emb = emb_ref[...]
    hk = jax.nn.silu(emb @ wk1_ref[...] + bk1_ref[...])
    hv = jax.nn.silu(emb @ wv1_ref[...] + bv1_ref[...])
    kw = _bmm(hk, wk2_ref[...]) + bk2_ref[...]
    vw = _bmm(hv, wv2_ref[...]) + bv2_ref[...]
    xse = xs @ tm_ref[...]
    k = ((xse * kw) @ sm_ref[...]) * isc
    v = ((xse * vw) @ sm_ref[...]) * isc
    qd = (xd @ wq_ref[...]) * isc
    kd = k @ bd_ref[...]
    logits = ((qd * kd) @ s2_ref[...]) * (1.0 / (DH * math.sqrt(DH)))
    ex = jnp.exp(logits)
    exr = ex @ e4_ref[...]
    out_ref[...] = jnp.concatenate([v * exr, exr], axis=1)


def _final_body(nf_ref, p0_ref, p1_ref, wo_ref, wf1_ref, wf2_ref, out_ref):
    isc = 1.0 / math.sqrt(C)
    s = p0_ref[0] + p1_ref[0]
    numer = s[:, :C]
    den = s[:, C:]
    agg = numer / (den + 1e-16)
    proj = (agg @ wo_ref[...]) * isc
    attn_out = nf_ref[...] + proj
    hid = (attn_out @ wf1_ref[...]) * isc
    act = hid * jax.nn.sigmoid(jnp.abs(hid))
    ffn = (act @ wf2_ref[...]) * (1.0 / math.sqrt(2 * C))
    out_ref[...] = attn_out + ffn


def kernel(node_features, edge_index, edge_sh, edge_radial_emb, W_q, Wk1, bk1,
           Wk2, bk2, Wv1, bv1, Wv2, bv2, Wd, W_o, W_f1, W_f2):
    src = edge_index[0]
    dst = edge_index[1]
    nf_b = node_features.astype(bf16)
    pad = EPH - HE

    def idx3(a, fill):
        return jnp.concatenate([a, jnp.full((pad,), fill, i32)]).reshape(
            NW, NCH, CHUNK)

    halves = []
    for h in (0, 1):
        s_h = lax.dynamic_slice(src, (h * HE,), (HE,))
        d_h = lax.dynamic_slice(dst, (h * HE,), (HE,))
        halves.append((idx3(s_h, 0), idx3(d_h, 0), idx3(d_h, N)))

    gather = pl.kernel(
        _gather_body,
        out_type=[jax.ShapeDtypeStruct((EPH, C), bf16),
                  jax.ShapeDtypeStruct((EPH, C), bf16)],
        mesh=_sc_mesh(),
        compiler_params=pltpu.CompilerParams(use_tc_tiling_on_sc=False),
        scratch_types=[pltpu.VMEM((NCH, CHUNK), i32), pltpu.VMEM((NCH, CHUNK), i32),
                       pltpu.VMEM((EPW, C), bf16), pltpu.VMEM((EPW, C), bf16),
                       pltpu.SemaphoreType.DMA, pltpu.SemaphoreType.DMA],
    )

    scatter = pl.kernel(
        _scatter_body,
        out_type=jax.ShapeDtypeStruct((2, NP, 2 * C), f32),
        mesh=_sc_mesh(),
        compiler_params=pltpu.CompilerParams(use_tc_tiling_on_sc=False),
        scratch_types=[pltpu.VMEM((NCH, CHUNK), i32),
                       pltpu.VMEM((7 * CHUNK, 2 * C), f32),
                       pltpu.VMEM_SHARED((NP, 2 * C), f32),
                       pltpu.SemaphoreType.DMA],
    )

    eye_c = jnp.eye(C, dtype=f32)
    tm = jnp.kron(eye_c, jnp.ones((1, C), f32))
    sm = jnp.kron(jnp.ones((C, 1), f32), eye_c)
    bd = jnp.kron(jnp.eye(H, dtype=f32), Wd.T)
    s2 = jnp.kron(jnp.eye(H, dtype=f32), jnp.ones((DH, 1), f32))
    e4 = s2.T

    def full(shape):
        return pl.BlockSpec(shape, lambda i: tuple(0 for _ in shape))

    def blk(shape, off=0):
        return pl.BlockSpec(shape, lambda i, off=off: (i + off,)
                            + tuple(0 for _ in shape[1:]))

    nblk = HE // EB

    def edge_half(h, xs_g, xd_g):
        return pl.pallas_call(
            _edge_body,
            grid=(nblk,),
            in_specs=[
                blk((EB, NB), h * nblk), blk((EB, 1), h * nblk),
                blk((EB, C)), blk((EB, C)),
                full((C, C)),
                full((NB, HID)), full((1, HID)), full((HID, C * C)),
                full((1, C * C)),
                full((NB, HID)), full((1, HID)), full((HID, C * C)),
                full((1, C * C)),
                full((C, C * C)), full((C * C, C)), full((C, C)), full((C, H)),
                full((H, C)),
            ],
            out_specs=blk((EB, 2 * C)),
            out_shape=jax.ShapeDtypeStruct((EPH, 2 * C), f32),
        )(edge_radial_emb, edge_sh, xs_g, xd_g, W_q,
          Wk1, bk1.reshape(1, HID), Wk2.astype(bf16), bk2.reshape(1, C * C),
          Wv1, bv1.reshape(1, HID), Wv2.astype(bf16), bv2.reshape(1, C * C),
          tm, sm, bd, s2, e4)

    xs_a, xd_a = gather(nf_b, halves[0][0], halves[0][1])
    xs_b, xd_b = gather(nf_b, halves[1][0], halves[1][1])
    pay_a = edge_half(0, xs_a, xd_a)
    pay_b = edge_half(1, xs_b, xd_b)
    zeros2 = jnp.zeros((2, NP, 2 * C), f32)
    parts_a = scatter(pay_a, halves[0][2], zeros2)
    parts = scatter(pay_b, halves[1][2], parts_a)

    def pblk(core):
        return pl.BlockSpec((1, NBK, 2 * C), lambda i, core=core: (core, i, 0))

    out = pl.pallas_call(
        _final_body,
        grid=(N // NBK,),
        in_specs=[
            blk((NBK, C)), pblk(0), pblk(1),
            full((C, C)), full((C, 2 * C)), full((2 * C, C)),
        ],
        out_specs=blk((NBK, C)),
        out_shape=jax.ShapeDtypeStruct((N, C), f32),
    )(node_features, parts, parts, W_o, W_f1, W_f2)
    return out


# speedup vs baseline: 1.1538x; 1.1538x over previous
import functools
import math

import jax
import jax.numpy as jnp
from jax import lax
from jax.experimental import pallas as pl
from jax.experimental.pallas import tpu as pltpu
from jax.experimental.pallas import tpu_sc as plsc

N = 10000
E = 100000
C = 32
H = 4
DH = C // H
NB = 16
HID = 64

NW = 32
CHUNK = 128
EPW = 3200
EP = NW * EPW
NCH = EPW // CHUNK
NP = 10240
RPS = NP // 16

SUP = 5
SROWS = SUP * CHUNK
NSUP = NCH // SUP

EB = 800
NBK = 1000

f32 = jnp.float32
bf16 = jnp.bfloat16
i32 = jnp.int32


def _sc_mesh():
    return plsc.VectorSubcoreMesh(core_axis_name="c", subcore_axis_name="s")


def _gather_body(nf_hbm, src3_hbm, dst3_hbm, xs_hbm, xd_hbm,
                 idx1, idx2, rows1, rows2, sem1, sem2):
    wid = lax.axis_index("s") * 2 + lax.axis_index("c")
    base = wid * EPW
    pltpu.sync_copy(src3_hbm.at[wid], idx1)
    pltpu.sync_copy(dst3_hbm.at[wid], idx2)

    def body(s, carry):
        off = base + s * SROWS
        cps = []
        for j in range(SUP):
            cps.append(pltpu.async_copy(
                nf_hbm.at[idx1.at[s * SUP + j]],
                rows1.at[pl.ds(j * CHUNK, CHUNK)], sem1))
            cps.append(pltpu.async_copy(
                nf_hbm.at[idx2.at[s * SUP + j]],
                rows2.at[pl.ds(j * CHUNK, CHUNK)], sem2))
        for cp in cps:
            cp.wait()
        pltpu.sync_copy(rows1, xs_hbm.at[pl.ds(off, SROWS)])
        pltpu.sync_copy(rows2, xd_hbm.at[pl.ds(off, SROWS)])
        return carry

    lax.fori_loop(0, NSUP, body, 0)


def _scatter_body(pay_hbm, dst3_hbm, zeros_hbm, part_hbm, idxb, payb, shared, sem):
    cid = lax.axis_index("c")
    sid = lax.axis_index("s")
    wid = sid * 2 + cid
    r0 = sid * RPS
    pltpu.sync_copy(zeros_hbm.at[pl.ds(r0, RPS)], shared.at[pl.ds(r0, RPS)])
    pltpu.sync_copy(dst3_hbm.at[wid], idxb)
    plsc.subcore_barrier()

    def body(s, carry):
        off = wid * EPW + s * SROWS
        pltpu.sync_copy(pay_hbm.at[pl.ds(off, SROWS)], payb)
        cps = []
        for j in range(SUP):
            cps.append(pltpu.async_copy(
                payb.at[pl.ds(j * CHUNK, CHUNK)],
                shared.at[idxb.at[s * SUP + j]], sem, add=True))
        for cp in cps:
            cp.wait()
        return carry

    lax.fori_loop(0, NSUP, body, 0)
    plsc.subcore_barrier()
    pltpu.sync_copy(shared.at[pl.ds(r0, RPS)], part_hbm.at[cid, pl.ds(r0, RPS)])


def _bmm(a, b):
    return lax.dot_general(a.astype(bf16), b.astype(bf16),
                           (((1,), (0,)), ((), ())),
                           preferred_element_type=f32)


def _edge_body(emb_ref, sh_ref, xs_ref, xd_ref, wq_ref,
               wk1_ref, bk1_ref, wk2_ref, bk2_ref,
               wv1_ref, bv1_ref, wv2_ref, bv2_ref,
               sm_ref, bd_ref, s2_ref, e4_ref, out_ref):
    isc = 1.0 / math.sqrt(C)
    xs = xs_ref[...].astype(f32) * sh_ref[...]
    xd = xd_ref[...].astype(f32)
    emb = emb_ref[...]
    hk = jax.nn.silu(emb @ wk1_ref[...] + bk1_ref[...])
    hv = jax.nn.silu(emb @ wv1_ref[...] + bv1_ref[...])
    kw = _bmm(hk, wk2_ref[...]) + bk2_ref[...]
    vw = _bmm(hv, wv2_ref[...]) + bv2_ref[...]
    xst = jnp.tile(xs, (1, C))
    k = ((xst * kw) @ sm_ref[...]) * isc
    v = ((xst * vw) @ sm_ref[...]) * isc
    qd = (xd @ wq_ref[...]) * isc
    kd = k @ bd_ref[...]
    logits = ((qd * kd) @ s2_ref[...]) * (1.0 / (DH * math.sqrt(DH)))
    ex = jnp.exp(logits)
    exr = ex @ e4_ref[...]
    out_ref[...] = jnp.concatenate([v * exr, exr], axis=1)


def _final_body(nf_ref, p0_ref, p1_ref, wo_ref, wf1_ref, wf2_ref, out_ref):
    isc = 1.0 / math.sqrt(C)
    s = p0_ref[0] + p1_ref[0]
    numer = s[:, :C]
    den = s[:, C:]
    agg = numer / (den + 1e-16)
    proj = (agg @ wo_ref[...]) * isc
    attn_out = nf_ref[...] + proj
    hid = (attn_out @ wf1_ref[...]) * isc
    act = hid * jax.nn.sigmoid(jnp.abs(hid))
    ffn = (act @ wf2_ref[...]) * (1.0 / math.sqrt(2 * C))
    out_ref[...] = attn_out + ffn


def kernel(node_features, edge_index, edge_sh, edge_radial_emb, W_q, Wk1, bk1,
           Wk2, bk2, Wv1, bv1, Wv2, bv2, Wd, W_o, W_f1, W_f2):
    src = edge_index[0]
    dst = edge_index[1]
    pad = EP - E
    src_3 = jnp.concatenate([src, jnp.zeros((pad,), i32)]).reshape(NW, NCH, CHUNK)
    dst_p = jnp.concatenate([dst, jnp.full((pad,), N, i32)])
    dst_3 = jnp.where(dst_p >= N, 0, dst_p).reshape(NW, NCH, CHUNK)
    dst_s3 = dst_p.reshape(NW, NCH, CHUNK)

    gather = pl.kernel(
        _gather_body,
        out_type=[jax.ShapeDtypeStruct((EP, C), bf16),
                  jax.ShapeDtypeStruct((EP, C), bf16)],
        mesh=_sc_mesh(),
        compiler_params=pltpu.CompilerParams(use_tc_tiling_on_sc=False),
        scratch_types=[pltpu.VMEM((NCH, CHUNK), i32), pltpu.VMEM((NCH, CHUNK), i32),
                       pltpu.VMEM((SROWS, C), bf16), pltpu.VMEM((SROWS, C), bf16),
                       pltpu.SemaphoreType.DMA, pltpu.SemaphoreType.DMA],
    )
    xs_g, xd_g = gather(node_features.astype(bf16), src_3, dst_3)

    wk2p = Wk2.reshape(HID, C, C).transpose(0, 2, 1).reshape(HID, C * C)
    wv2p = Wv2.reshape(HID, C, C).transpose(0, 2, 1).reshape(HID, C * C)
    bk2p = bk2.reshape(C, C).T.reshape(1, C * C)
    bv2p = bv2.reshape(C, C).T.reshape(1, C * C)
    sm2 = jnp.kron(jnp.eye(C, dtype=f32), jnp.ones((C, 1), f32))
    bd = jnp.kron(jnp.eye(H, dtype=f32), Wd.T)
    s2 = jnp.kron(jnp.eye(H, dtype=f32), jnp.ones((DH, 1), f32))
    e4 = s2.T

    def full(shape):
        return pl.BlockSpec(shape, lambda i: tuple(0 for _ in shape))

    def blk(shape):
        return pl.BlockSpec(shape, lambda i: (i,) + tuple(0 for _ in shape[1:]))

    payload = pl.pallas_call(
        _edge_body,
        grid=(E // EB,),
        in_specs=[
            blk((EB, NB)), blk((EB, 1)), blk((EB, C)), blk((EB, C)),
            full((C, C)),
            full((NB, HID)), full((1, HID)), full((HID, C * C)), full((1, C * C)),
            full((NB, HID)), full((1, HID)), full((HID, C * C)), full((1, C * C)),
            full((C * C, C)), full((C, C)), full((C, H)),
            full((H, C)),
        ],
        out_specs=blk((EB, 2 * C)),
        out_shape=jax.ShapeDtypeStruct((EP, 2 * C), f32),
    )(edge_radial_emb, edge_sh, xs_g, xd_g, W_q,
      Wk1, bk1.reshape(1, HID), wk2p.astype(bf16), bk2p,
      Wv1, bv1.reshape(1, HID), wv2p.astype(bf16), bv2p,
      sm2, bd, s2, e4)

    zeros_acc = jnp.zeros((NP, 2 * C), f32)
    scatter = pl.kernel(
        _scatter_body,
        out_type=jax.ShapeDtypeStruct((2, NP, 2 * C), f32),
        mesh=_sc_mesh(),
        compiler_params=pltpu.CompilerParams(use_tc_tiling_on_sc=False),
        scratch_types=[pltpu.VMEM((NCH, CHUNK), i32),
                       pltpu.VMEM((SROWS, 2 * C), f32),
                       pltpu.VMEM_SHARED((NP, 2 * C), f32),
                       pltpu.SemaphoreType.DMA],
    )
    parts = scatter(payload, dst_s3, zeros_acc)

    def pblk(core):
        return pl.BlockSpec((1, NBK, 2 * C), lambda i, core=core: (core, i, 0))

    out = pl.pallas_call(
        _final_body,
        grid=(N // NBK,),
        in_specs=[
            blk((NBK, C)), pblk(0), pblk(1),
            full((C, C)), full((C, 2 * C)), full((2 * C, C)),
        ],
        out_specs=blk((NBK, C)),
        out_shape=jax.ShapeDtypeStruct((N, C), f32),
    )(node_features, parts, parts, W_o, W_f1, W_f2)
    return out


# speedup vs baseline: 1.2301x; 1.0661x over previous
import functools
import math

import jax
import jax.numpy as jnp
from jax import lax
from jax.experimental import pallas as pl
from jax.experimental.pallas import tpu as pltpu
from jax.experimental.pallas import tpu_sc as plsc

N = 10000
E = 100000
C = 32
H = 4
DH = C // H
NB = 16
HID = 64

NW = 32
CHUNK = 128
EPW = 3200
EP = NW * EPW
NCH = EPW // CHUNK
NP = 10240
RPS = NP // 16

SUP = 5
SROWS = SUP * CHUNK
NSUP = NCH // SUP

EB = 2000
NBK = 1000

f32 = jnp.float32
bf16 = jnp.bfloat16
i32 = jnp.int32


def _sc_mesh():
    return plsc.VectorSubcoreMesh(core_axis_name="c", subcore_axis_name="s")


def _gather_body(nf_hbm, src3_hbm, dst3_hbm, xs_hbm, xd_hbm,
                 idx1, idx2, rows1, rows2, sem1, sem2):
    wid = lax.axis_index("s") * 2 + lax.axis_index("c")
    base = wid * EPW
    pltpu.sync_copy(src3_hbm.at[wid], idx1)
    pltpu.sync_copy(dst3_hbm.at[wid], idx2)

    def body(s, carry):
        off = base + s * SROWS
        cps = []
        for j in range(SUP):
            cps.append(pltpu.async_copy(
                nf_hbm.at[idx1.at[s * SUP + j]],
                rows1.at[pl.ds(j * CHUNK, CHUNK)], sem1))
            cps.append(pltpu.async_copy(
                nf_hbm.at[idx2.at[s * SUP + j]],
                rows2.at[pl.ds(j * CHUNK, CHUNK)], sem2))
        for cp in cps:
            cp.wait()
        pltpu.sync_copy(rows1, xs_hbm.at[pl.ds(off, SROWS)])
        pltpu.sync_copy(rows2, xd_hbm.at[pl.ds(off, SROWS)])
        return carry

    lax.fori_loop(0, NSUP, body, 0)


def _scatter_body(pay_hbm, dst3_hbm, zeros_hbm, part_hbm, idxb, payb, shared, sem):
    cid = lax.axis_index("c")
    sid = lax.axis_index("s")
    wid = sid * 2 + cid
    r0 = sid * RPS
    pltpu.sync_copy(zeros_hbm.at[pl.ds(r0, RPS)], shared.at[pl.ds(r0, RPS)])
    pltpu.sync_copy(dst3_hbm.at[wid], idxb)
    plsc.subcore_barrier()

    def body(s, carry):
        off = wid * EPW + s * SROWS
        pltpu.sync_copy(pay_hbm.at[pl.ds(off, SROWS)], payb)
        cps = []
        for j in range(SUP):
            cps.append(pltpu.async_copy(
                payb.at[pl.ds(j * CHUNK, CHUNK)],
                shared.at[idxb.at[s * SUP + j]], sem, add=True))
        for cp in cps:
            cp.wait()
        return carry

    lax.fori_loop(0, NSUP, body, 0)
    plsc.subcore_barrier()
    pltpu.sync_copy(shared.at[pl.ds(r0, RPS)], part_hbm.at[cid, pl.ds(r0, RPS)])


def _bmm(a, b):
    return lax.dot_general(a.astype(bf16), b.astype(bf16),
                           (((1,), (0,)), ((), ())),
                           preferred_element_type=f32)


def _edge_body(emb_ref, sh_ref, xs_ref, xd_ref, wq_ref,
               wk1_ref, bk1_ref, wk2_ref, bk2_ref,
               wv1_ref, bv1_ref, wv2_ref, bv2_ref,
               sm_ref, bd_ref, s2_ref, e4_ref, out_ref):
    isc = 1.0 / math.sqrt(C)
    xs = xs_ref[...].astype(f32) * sh_ref[...]
    xd = xd_ref[...].astype(f32)
    emb = emb_ref[...]
    hk = jax.nn.silu(emb @ wk1_ref[...] + bk1_ref[...])
    hv = jax.nn.silu(emb @ wv1_ref[...] + bv1_ref[...])
    kw = _bmm(hk, wk2_ref[...]) + bk2_ref[...]
    vw = _bmm(hv, wv2_ref[...]) + bv2_ref[...]
    xst = jnp.tile(xs, (1, C))
    k = ((xst * kw) @ sm_ref[...]) * isc
    v = ((xst * vw) @ sm_ref[...]) * isc
    qd = (xd @ wq_ref[...]) * isc
    kd = k @ bd_ref[...]
    logits = ((qd * kd) @ s2_ref[...]) * (1.0 / (DH * math.sqrt(DH)))
    ex = jnp.exp(logits)
    exr = ex @ e4_ref[...]
    out_ref[...] = jnp.concatenate([v * exr, exr], axis=1)


def _final_body(nf_ref, p0_ref, p1_ref, wo_ref, wf1_ref, wf2_ref, out_ref):
    isc = 1.0 / math.sqrt(C)
    s = p0_ref[0] + p1_ref[0]
    numer = s[:, :C]
    den = s[:, C:]
    agg = numer / (den + 1e-16)
    proj = (agg @ wo_ref[...]) * isc
    attn_out = nf_ref[...] + proj
    hid = (attn_out @ wf1_ref[...]) * isc
    act = hid * jax.nn.sigmoid(jnp.abs(hid))
    ffn = (act @ wf2_ref[...]) * (1.0 / math.sqrt(2 * C))
    out_ref[...] = attn_out + ffn


def kernel(node_features, edge_index, edge_sh, edge_radial_emb, W_q, Wk1, bk1,
           Wk2, bk2, Wv1, bv1, Wv2, bv2, Wd, W_o, W_f1, W_f2):
    src = edge_index[0]
    dst = edge_index[1]
    pad = EP - E
    src_3 = jnp.concatenate([src, jnp.zeros((pad,), i32)]).reshape(NW, NCH, CHUNK)
    dst_p = jnp.concatenate([dst, jnp.full((pad,), N, i32)])
    dst_3 = jnp.where(dst_p >= N, 0, dst_p).reshape(NW, NCH, CHUNK)
    dst_s3 = dst_p.reshape(NW, NCH, CHUNK)

    gather = pl.kernel(
        _gather_body,
        out_type=[jax.ShapeDtypeStruct((EP, C), bf16),
                  jax.ShapeDtypeStruct((EP, C), bf16)],
        mesh=_sc_mesh(),
        compiler_params=pltpu.CompilerParams(use_tc_tiling_on_sc=False),
        scratch_types=[pltpu.VMEM((NCH, CHUNK), i32), pltpu.VMEM((NCH, CHUNK), i32),
                       pltpu.VMEM((SROWS, C), bf16), pltpu.VMEM((SROWS, C), bf16),
                       pltpu.SemaphoreType.DMA, pltpu.SemaphoreType.DMA],
    )
    xs_g, xd_g = gather(node_features.astype(bf16), src_3, dst_3)

    wk2p = Wk2.reshape(HID, C, C).transpose(0, 2, 1).reshape(HID, C * C)
    wv2p = Wv2.reshape(HID, C, C).transpose(0, 2, 1).reshape(HID, C * C)
    bk2p = bk2.reshape(C, C).T.reshape(1, C * C)
    bv2p = bv2.reshape(C, C).T.reshape(1, C * C)
    sm2 = jnp.kron(jnp.eye(C, dtype=f32), jnp.ones((C, 1), f32))
    bd = jnp.kron(jnp.eye(H, dtype=f32), Wd.T)
    s2 = jnp.kron(jnp.eye(H, dtype=f32), jnp.ones((DH, 1), f32))
    e4 = s2.T

    def full(shape):
        return pl.BlockSpec(shape, lambda i: tuple(0 for _ in shape))

    def blk(shape):
        return pl.BlockSpec(shape, lambda i: (i,) + tuple(0 for _ in shape[1:]))

    payload = pl.pallas_call(
        _edge_body,
        grid=(E // EB,),
        in_specs=[
            blk((EB, NB)), blk((EB, 1)), blk((EB, C)), blk((EB, C)),
            full((C, C)),
            full((NB, HID)), full((1, HID)), full((HID, C * C)), full((1, C * C)),
            full((NB, HID)), full((1, HID)), full((HID, C * C)), full((1, C * C)),
            full((C * C, C)), full((C, C)), full((C, H)),
            full((H, C)),
        ],
        out_specs=blk((EB, 2 * C)),
        out_shape=jax.ShapeDtypeStruct((EP, 2 * C), f32),
    )(edge_radial_emb, edge_sh, xs_g, xd_g, W_q,
      Wk1, bk1.reshape(1, HID), wk2p.astype(bf16), bk2p,
      Wv1, bv1.reshape(1, HID), wv2p.astype(bf16), bv2p,
      sm2, bd, s2, e4)

    zeros_acc = jnp.zeros((NP, 2 * C), f32)
    scatter = pl.kernel(
        _scatter_body,
        out_type=jax.ShapeDtypeStruct((2, NP, 2 * C), f32),
        mesh=_sc_mesh(),
        compiler_params=pltpu.CompilerParams(use_tc_tiling_on_sc=False),
        scratch_types=[pltpu.VMEM((NCH, CHUNK), i32),
                       pltpu.VMEM((SROWS, 2 * C), f32),
                       pltpu.VMEM_SHARED((NP, 2 * C), f32),
                       pltpu.SemaphoreType.DMA],
    )
    parts = scatter(payload, dst_s3, zeros_acc)

    def pblk(core):
        return pl.BlockSpec((1, NBK, 2 * C), lambda i, core=core: (core, i, 0))

    out = pl.pallas_call(
        _final_body,
        grid=(N // NBK,),
        in_specs=[
            blk((NBK, C)), pblk(0), pblk(1),
            full((C, C)), full((C, 2 * C)), full((2 * C, C)),
        ],
        out_specs=blk((NBK, C)),
        out_shape=jax.ShapeDtypeStruct((N, C), f32),
    )(node_features, parts, parts, W_o, W_f1, W_f2)
    return out


# speedup vs baseline: 1.2441x; 1.0114x over previous
import functools
import math

import jax
import jax.numpy as jnp
from jax import lax
from jax.experimental import pallas as pl
from jax.experimental.pallas import tpu as pltpu
from jax.experimental.pallas import tpu_sc as plsc

N = 10000
E = 100000
C = 32
H = 4
DH = C // H
NB = 16
HID = 64

NW = 32
CHUNK = 128
EPW = 3200
EP = NW * EPW
NCH = EPW // CHUNK
NP = 10240
RPS = NP // 16

SUP = 5
SROWS = SUP * CHUNK
NSUP = NCH // SUP

EB = 4000
NBK = 1000

f32 = jnp.float32
bf16 = jnp.bfloat16
i32 = jnp.int32


def _sc_mesh():
    return plsc.VectorSubcoreMesh(core_axis_name="c", subcore_axis_name="s")


def _gather_body(nf_hbm, src3_hbm, dst3_hbm, xs_hbm, xd_hbm,
                 idx1, idx2, rows1, rows2, sem1, sem2):
    wid = lax.axis_index("s") * 2 + lax.axis_index("c")
    base = wid * EPW
    pltpu.sync_copy(src3_hbm.at[wid], idx1)
    pltpu.sync_copy(dst3_hbm.at[wid], idx2)

    def body(s, carry):
        off = base + s * SROWS
        cps = []
        for j in range(SUP):
            cps.append(pltpu.async_copy(
                nf_hbm.at[idx1.at[s * SUP + j]],
                rows1.at[pl.ds(j * CHUNK, CHUNK)], sem1))
            cps.append(pltpu.async_copy(
                nf_hbm.at[idx2.at[s * SUP + j]],
                rows2.at[pl.ds(j * CHUNK, CHUNK)], sem2))
        for cp in cps:
            cp.wait()
        pltpu.sync_copy(rows1, xs_hbm.at[pl.ds(off, SROWS)])
        pltpu.sync_copy(rows2, xd_hbm.at[pl.ds(off, SROWS)])
        return carry

    lax.fori_loop(0, NSUP, body, 0)


def _scatter_body(pay_hbm, dst3_hbm, zeros_hbm, part_hbm, idxb, payb, shared, sem):
    cid = lax.axis_index("c")
    sid = lax.axis_index("s")
    wid = sid * 2 + cid
    r0 = sid * RPS
    pltpu.sync_copy(zeros_hbm.at[pl.ds(r0, RPS)], shared.at[pl.ds(r0, RPS)])
    pltpu.sync_copy(dst3_hbm.at[wid], idxb)
    plsc.subcore_barrier()

    def body(s, carry):
        off = wid * EPW + s * SROWS
        pltpu.sync_copy(pay_hbm.at[pl.ds(off, SROWS)], payb)
        cps = []
        for j in range(SUP):
            cps.append(pltpu.async_copy(
                payb.at[pl.ds(j * CHUNK, CHUNK)],
                shared.at[idxb.at[s * SUP + j]], sem, add=True))
        for cp in cps:
            cp.wait()
        return carry

    lax.fori_loop(0, NSUP, body, 0)
    plsc.subcore_barrier()
    pltpu.sync_copy(shared.at[pl.ds(r0, RPS)], part_hbm.at[cid, pl.ds(r0, RPS)])


def _bmm(a, b):
    return lax.dot_general(a.astype(bf16), b.astype(bf16),
                           (((1,), (0,)), ((), ())),
                           preferred_element_type=f32)


def _edge_body(emb_ref, sh_ref, xs_ref, xd_ref, wq_ref,
               wk1_ref, bk1_ref, wk2_ref, bk2_ref,
               wv1_ref, bv1_ref, wv2_ref, bv2_ref,
               sm_ref, bd_ref, s2_ref, e4_ref, out_ref):
    isc = 1.0 / math.sqrt(C)
    xs = xs_ref[...].astype(f32) * sh_ref[...]
    xd = xd_ref[...].astype(f32)
    emb = emb_ref[...]
    hk = jax.nn.silu(emb @ wk1_ref[...] + bk1_ref[...])
    hv = jax.nn.silu(emb @ wv1_ref[...] + bv1_ref[...])
    kw = _bmm(hk, wk2_ref[...]) + bk2_ref[...]
    vw = _bmm(hv, wv2_ref[...]) + bv2_ref[...]
    xst = jnp.tile(xs, (1, C))
    k = ((xst * kw) @ sm_ref[...]) * isc
    v = ((xst * vw) @ sm_ref[...]) * isc
    qd = (xd @ wq_ref[...]) * isc
    kd = k @ bd_ref[...]
    logits = ((qd * kd) @ s2_ref[...]) * (1.0 / (DH * math.sqrt(DH)))
    ex = jnp.exp(logits)
    exr = ex @ e4_ref[...]
    out_ref[...] = jnp.concatenate([v * exr, exr], axis=1)


def _final_body(nf_ref, p0_ref, p1_ref, wo_ref, wf1_ref, wf2_ref, out_ref):
    isc = 1.0 / math.sqrt(C)
    s = p0_ref[0] + p1_ref[0]
    numer = s[:, :C]
    den = s[:, C:]
    agg = numer / (den + 1e-16)
    proj = (agg @ wo_ref[...]) * isc
    attn_out = nf_ref[...] + proj
    hid = (attn_out @ wf1_ref[...]) * isc
    act = hid * jax.nn.sigmoid(jnp.abs(hid))
    ffn = (act @ wf2_ref[...]) * (1.0 / math.sqrt(2 * C))
    out_ref[...] = attn_out + ffn


def kernel(node_features, edge_index, edge_sh, edge_radial_emb, W_q, Wk1, bk1,
           Wk2, bk2, Wv1, bv1, Wv2, bv2, Wd, W_o, W_f1, W_f2):
    src = edge_index[0]
    dst = edge_index[1]
    pad = EP - E
    src_3 = jnp.concatenate([src, jnp.zeros((pad,), i32)]).reshape(NW, NCH, CHUNK)
    dst_p = jnp.concatenate([dst, jnp.full((pad,), N, i32)])
    dst_3 = jnp.where(dst_p >= N, 0, dst_p).reshape(NW, NCH, CHUNK)
    dst_s3 = dst_p.reshape(NW, NCH, CHUNK)

    gather = pl.kernel(
        _gather_body,
        out_type=[jax.ShapeDtypeStruct((EP, C), bf16),
                  jax.ShapeDtypeStruct((EP, C), bf16)],
        mesh=_sc_mesh(),
        compiler_params=pltpu.CompilerParams(use_tc_tiling_on_sc=False),
        scratch_types=[pltpu.VMEM((NCH, CHUNK), i32), pltpu.VMEM((NCH, CHUNK), i32),
                       pltpu.VMEM((SROWS, C), bf16), pltpu.VMEM((SROWS, C), bf16),
                       pltpu.SemaphoreType.DMA, pltpu.SemaphoreType.DMA],
    )
    xs_g, xd_g = gather(node_features.astype(bf16), src_3, dst_3)

    wk2p = Wk2.reshape(HID, C, C).transpose(0, 2, 1).reshape(HID, C * C)
    wv2p = Wv2.reshape(HID, C, C).transpose(0, 2, 1).reshape(HID, C * C)
    bk2p = bk2.reshape(C, C).T.reshape(1, C * C)
    bv2p = bv2.reshape(C, C).T.reshape(1, C * C)
    sm2 = jnp.kron(jnp.eye(C, dtype=f32), jnp.ones((C, 1), f32))
    bd = jnp.kron(jnp.eye(H, dtype=f32), Wd.T)
    s2 = jnp.kron(jnp.eye(H, dtype=f32), jnp.ones((DH, 1), f32))
    e4 = s2.T

    def full(shape):
        return pl.BlockSpec(shape, lambda i: tuple(0 for _ in shape))

    def blk(shape):
        return pl.BlockSpec(shape, lambda i: (i,) + tuple(0 for _ in shape[1:]))

    payload = pl.pallas_call(
        _edge_body,
        grid=(E // EB,),
        in_specs=[
            blk((EB, NB)), blk((EB, 1)), blk((EB, C)), blk((EB, C)),
            full((C, C)),
            full((NB, HID)), full((1, HID)), full((HID, C * C)), full((1, C * C)),
            full((NB, HID)), full((1, HID)), full((HID, C * C)), full((1, C * C)),
            full((C * C, C)), full((C, C)), full((C, H)),
            full((H, C)),
        ],
        out_specs=blk((EB, 2 * C)),
        out_shape=jax.ShapeDtypeStruct((EP, 2 * C), f32),
    )(edge_radial_emb, edge_sh, xs_g, xd_g, W_q,
      Wk1, bk1.reshape(1, HID), wk2p.astype(bf16), bk2p,
      Wv1, bv1.reshape(1, HID), wv2p.astype(bf16), bv2p,
      sm2, bd, s2, e4)

    zeros_acc = jnp.zeros((NP, 2 * C), f32)
    scatter = pl.kernel(
        _scatter_body,
        out_type=jax.ShapeDtypeStruct((2, NP, 2 * C), f32),
        mesh=_sc_mesh(),
        compiler_params=pltpu.CompilerParams(use_tc_tiling_on_sc=False),
        scratch_types=[pltpu.VMEM((NCH, CHUNK), i32),
                       pltpu.VMEM((SROWS, 2 * C), f32),
                       pltpu.VMEM_SHARED((NP, 2 * C), f32),
                       pltpu.SemaphoreType.DMA],
    )
    parts = scatter(payload, dst_s3, zeros_acc)

    def pblk(core):
        return pl.BlockSpec((1, NBK, 2 * C), lambda i, core=core: (core, i, 0))

    out = pl.pallas_call(
        _final_body,
        grid=(N // NBK,),
        in_specs=[
            blk((NBK, C)), pblk(0), pblk(1),
            full((C, C)), full((C, 2 * C)), full((2 * C, C)),
        ],
        out_specs=blk((NBK, C)),
        out_shape=jax.ShapeDtypeStruct((N, C), f32),
    )(node_features, parts, parts, W_o, W_f1, W_f2)
    return out
